# baseline (speedup 1.0000x reference)
import jax
import jax.numpy as jnp
from jax.experimental import pallas as pl
from jax.experimental.pallas import tpu as pltpu


def _adapter_body(x_ref, wd_ref, bd_ref, wu_ref, bu_ref, scale_ref, o_ref):
    s = scale_ref[0]
    x = x_ref[...].astype(jnp.bfloat16)
    down = jnp.dot(x, wd_ref[...].astype(jnp.bfloat16),
                   preferred_element_type=jnp.float32)
    down = jnp.maximum(down + bd_ref[...], 0.0) * s
    up = jnp.dot(down.astype(jnp.bfloat16), wu_ref[...].astype(jnp.bfloat16),
                 preferred_element_type=jnp.float32)
    o_ref[...] = up + bu_ref[...] * s


def kernel(x, wd_t, bd, wu_t, bu, scale):
    B, S, D = x.shape
    Rp = wd_t.shape[1]
    M = B * S
    x2 = x.reshape(M, D)

    TM = 2048
    while TM > 8 and M % TM != 0:
        TM //= 2
    steps = M // TM

    def resident(shape):
        return pl.BlockSpec(shape, lambda i: (0, 0))

    out2 = pl.pallas_call(
        _adapter_body,
        out_shape=jax.ShapeDtypeStruct((M, D), x.dtype),
        grid=(steps,),
        in_specs=[
            pl.BlockSpec((TM, D), lambda i: (i, 0)),
            resident((D, Rp)),
            resident((1, Rp)),
            resident((Rp, D)),
            resident((1, D)),
            pl.BlockSpec(memory_space=pltpu.MemorySpace.SMEM),
        ],
        out_specs=pl.BlockSpec((TM, D), lambda i: (i, 0)),
        compiler_params=pltpu.CompilerParams(
            dimension_semantics=("parallel",),
            vmem_limit_bytes=48 * 1024 * 1024),
    )(x2, wd_t, bd.astype(jnp.float32), wu_t, bu.astype(jnp.float32),
      scale.astype(jnp.float32).reshape(1))

    return out2.reshape(B, S, D)
